# baseline (speedup 1.0000x reference)
import jax
import jax.numpy as jnp
from jax import lax
from jax.experimental import pallas as pl
from jax.experimental.pallas import tpu as pltpu
from jax.experimental.pallas import tpu_sc as plsc

N = 10000
D = 128
E = 320000
R = 3
NC = 2
NS = 16
NW = NC * NS
ET = R * E
EPW = ET // NW
K = 80
NWIN = EPW // K
UNROLL = 3
NP = 10240
RPT = NP // NS
ZR = 64

_DN = (((1,), (1,)), ((), ()))


def _mm_body(x_ref, w0_ref, w1_ref, w2_ref, wr_ref, b_ref,
             h0_ref, h1_ref, h2_ref, xr_ref):
    x = x_ref[...]
    h0_ref[...] = lax.dot_general(x, w0_ref[...], _DN, preferred_element_type=jnp.float32)
    h1_ref[...] = lax.dot_general(x, w1_ref[...], _DN, preferred_element_type=jnp.float32)
    h2_ref[...] = lax.dot_general(x, w2_ref[...], _DN, preferred_element_type=jnp.float32)
    xr_ref[...] = lax.dot_general(x, wr_ref[...], _DN, preferred_element_type=jnp.float32) + b_ref[...]


_BM = 2000

_mm_call = pl.pallas_call(
    _mm_body,
    grid=(N // _BM,),
    in_specs=[pl.BlockSpec((_BM, D), lambda i: (i, 0))]
    + [pl.BlockSpec((D, D), lambda i: (0, 0))] * 4
    + [pl.BlockSpec((1, D), lambda i: (0, 0))],
    out_specs=[pl.BlockSpec((_BM, D), lambda i: (i, 0))] * 4,
    out_shape=[jax.ShapeDtypeStruct((N, D), jnp.float32)] * 4,
)


def _combine_body(p0_ref, p1_ref, o_ref):
    o_ref[...] = p0_ref[...] + p1_ref[...]


_combine_call = pl.pallas_call(
    _combine_body,
    grid=(N // _BM,),
    in_specs=[pl.BlockSpec((_BM, D), lambda i: (i, 0)),
              pl.BlockSpec((_BM, D), lambda i: (i, 0))],
    out_specs=pl.BlockSpec((_BM, D), lambda i: (i, 0)),
    out_shape=jax.ShapeDtypeStruct((N, D), jnp.float32),
)


def _sc_body(e0f, e1f, e2f, h0, h1, h2, xroot,
             out0, out1,
             acc,
             sidx0, sidx1, sidx2, didx0, didx1, didx2,
             rows0, rows1, rows2, zbuf,
             semi0, semi1, semi2, semg0, semg1, semg2):
    c = lax.axis_index("c")
    s = lax.axis_index("s")
    wid = s * NC + c

    sidx = (sidx0, sidx1, sidx2)
    didx = (didx0, didx1, didx2)
    rows = (rows0, rows1, rows2)
    semi = (semi0, semi1, semi2)
    semg = (semg0, semg1, semg2)

    def rel_branch(w, fn):
        base = wid * EPW + w * K

        @pl.when(base < E)
        def _():
            fn(e0f, h0, base)

        @pl.when(jnp.logical_and(base >= E, base < 2 * E))
        def _():
            fn(e1f, h1, base - E)

        @pl.when(base >= 2 * E)
        def _():
            fn(e2f, h2, base - 2 * E)

    def idx_issue(w, slot):
        def go(ef, h, off):
            pltpu.async_copy(ef.at[pl.ds(off, K)], sidx[slot], semi[slot])
            pltpu.async_copy(ef.at[pl.ds(E + off, K)], didx[slot], semi[slot])
        rel_branch(w, go)

    def idx_wait(slot):
        pltpu.make_async_copy(e0f.at[pl.ds(0, K)], sidx[slot], semi[slot]).wait()
        pltpu.make_async_copy(e0f.at[pl.ds(0, K)], didx[slot], semi[slot]).wait()

    def g_issue(w, slot):
        def go(ef, h, off):
            pltpu.async_copy(h.at[sidx[slot]], rows[slot], semg[slot])
        rel_branch(w, go)

    def g_wait(slot):
        pltpu.make_async_copy(h0.at[pl.ds(0, K)], rows[slot], semg[slot]).wait()

    def scatter(slot):
        pltpu.sync_copy(rows[slot], acc.at[didx[slot]], add=True)

    zbase = s * RPT
    for w in range(UNROLL):
        idx_issue(w, w)

    _TAIL = N - (NS - 1) * RPT
    use_root = jnp.where(c == 0, s < NS // 2, s >= NS // 2)

    @pl.when(jnp.logical_and(use_root, s < NS - 1))
    def _():
        pltpu.async_copy(xroot.at[pl.ds(zbase, RPT), :],
                         acc.at[pl.ds(zbase, RPT), :], semg0).wait()

    @pl.when(jnp.logical_and(use_root, s == NS - 1))
    def _():
        pltpu.async_copy(xroot.at[pl.ds(zbase, _TAIL), :],
                         acc.at[pl.ds(zbase, _TAIL), :], semg0).wait()

    @pl.when(jnp.logical_not(use_root))
    def _():
        z16 = jnp.zeros((16,), jnp.float32)

        def _zrow(i, carry):
            for j in range(D // 16):
                zbuf[i, pl.ds(j * 16, 16)] = z16
            return carry

        lax.fori_loop(0, ZR, _zrow, 0)
        for i in range(RPT // ZR):
            pltpu.sync_copy(zbuf, acc.at[pl.ds(zbase + i * ZR, ZR), :])

    plsc.subcore_barrier()

    idx_wait(0)
    g_issue(0, 0)
    idx_wait(1)
    g_issue(1, 1)

    def _body(w3, carry):
        w = UNROLL * w3
        for j in range(UNROLL):
            jn = (j + 2) % UNROLL
            g_wait(j)

            @pl.when(w + j + 2 < NWIN)
            def _():
                idx_wait(jn)
                g_issue(w + j + 2, jn)

            scatter(j)

            @pl.when(w + j + UNROLL < NWIN)
            def _():
                idx_issue(w + j + UNROLL, j)

        return carry

    lax.fori_loop(0, NWIN // UNROLL, _body, 0)
    plsc.subcore_barrier()

    obase = s * RPT
    out = (out0, out1)
    for cc in range(NC):
        @pl.when(jnp.logical_and(c == cc, s < NS - 1))
        def _(cc=cc):
            pltpu.sync_copy(acc.at[pl.ds(obase, RPT), :],
                            out[cc].at[pl.ds(obase, RPT), :])

        @pl.when(jnp.logical_and(c == cc, s == NS - 1))
        def _(cc=cc):
            pltpu.sync_copy(acc.at[pl.ds(obase, _TAIL), :],
                            out[cc].at[pl.ds(obase, _TAIL), :])


def _make_sc_call():
    return pl.kernel(
        _sc_body,
        out_type=(jax.ShapeDtypeStruct((NP, D), jnp.float32),) * 2,
        mesh=plsc.VectorSubcoreMesh(core_axis_name="c", subcore_axis_name="s"),
        scratch_types=[
            pltpu.VMEM_SHARED((NP, D), jnp.float32),
        ]
        + [pltpu.VMEM((K,), jnp.int32)] * 6
        + [pltpu.VMEM((K, D), jnp.float32)] * 3
        + [pltpu.VMEM((ZR, D), jnp.float32)]
        + [pltpu.SemaphoreType.DMA] * 6,
    )


def kernel(x, edge_index_0, edge_index_1, edge_index_2, W0, W1, W2, W_root, b_root):
    h0, h1, h2, xroot = _mm_call(x, W0, W1, W2, W_root, b_root.reshape(1, D))
    sc = _make_sc_call()
    p0, p1 = sc(edge_index_0.reshape(2 * E), edge_index_1.reshape(2 * E),
                edge_index_2.reshape(2 * E), h0, h1, h2, xroot)
    return _combine_call(p0, p1)


# speedup vs baseline: 1.1590x; 1.1590x over previous
import jax
import jax.numpy as jnp
from jax import lax
from jax.experimental import pallas as pl
from jax.experimental.pallas import tpu as pltpu
from jax.experimental.pallas import tpu_sc as plsc

N = 10000
D = 128
E = 320000
R = 3
NC = 2
NS = 16
NW = NC * NS
ET = R * E
EPW = ET // NW
K = 80
NWIN = EPW // K
UNROLL = 3
NP = 10240
RPT = NP // NS
ZR = 64

_DN = (((1,), (1,)), ((), ()))


def _mm_body(x_ref, w0_ref, w1_ref, w2_ref, wr_ref, b_ref,
             h0_ref, h1_ref, h2_ref, xr_ref):
    x = x_ref[...]
    h0_ref[...] = lax.dot_general(x, w0_ref[...], _DN, preferred_element_type=jnp.float32)
    h1_ref[...] = lax.dot_general(x, w1_ref[...], _DN, preferred_element_type=jnp.float32)
    h2_ref[...] = lax.dot_general(x, w2_ref[...], _DN, preferred_element_type=jnp.float32)
    xr_ref[...] = lax.dot_general(x, wr_ref[...], _DN, preferred_element_type=jnp.float32) + b_ref[...]


_BM = 2000

_mm_call = pl.pallas_call(
    _mm_body,
    grid=(N // _BM,),
    in_specs=[pl.BlockSpec((_BM, D), lambda i: (i, 0))]
    + [pl.BlockSpec((D, D), lambda i: (0, 0))] * 4
    + [pl.BlockSpec((1, D), lambda i: (0, 0))],
    out_specs=[pl.BlockSpec((_BM, D), lambda i: (i, 0))] * 4,
    out_shape=[jax.ShapeDtypeStruct((N, D), jnp.float32)] * 4,
)


def _combine_body(p0_ref, p1_ref, o_ref):
    o_ref[...] = p0_ref[...] + p1_ref[...]


_combine_call = pl.pallas_call(
    _combine_body,
    grid=(N // _BM,),
    in_specs=[pl.BlockSpec((_BM, D), lambda i: (i, 0)),
              pl.BlockSpec((_BM, D), lambda i: (i, 0))],
    out_specs=pl.BlockSpec((_BM, D), lambda i: (i, 0)),
    out_shape=jax.ShapeDtypeStruct((N, D), jnp.float32),
)


def _sc_body(e0f, e1f, e2f, h0, h1, h2, xroot,
             out0, out1,
             acc,
             sidx0, sidx1, sidx2, didx0, didx1, didx2,
             rows0, rows1, rows2, zbuf,
             semi0, semi1, semi2, semg0, semg1, semg2):
    c = lax.axis_index("c")
    s = lax.axis_index("s")
    wid = s * NC + c

    sidx = (sidx0, sidx1, sidx2)
    didx = (didx0, didx1, didx2)
    rows = (rows0, rows1, rows2)
    semi = (semi0, semi1, semi2)
    semg = (semg0, semg1, semg2)

    def rel_branch(w, fn):
        base = wid * EPW + w * K

        @pl.when(base < E)
        def _():
            fn(e0f, h0, base)

        @pl.when(jnp.logical_and(base >= E, base < 2 * E))
        def _():
            fn(e1f, h1, base - E)

        @pl.when(base >= 2 * E)
        def _():
            fn(e2f, h2, base - 2 * E)

    def idx_issue(w, slot):
        def go(ef, h, off):
            pltpu.async_copy(ef.at[pl.ds(off, K)], sidx[slot], semi[slot])
            pltpu.async_copy(ef.at[pl.ds(E + off, K)], didx[slot], semi[slot])
        rel_branch(w, go)

    def idx_wait(slot):
        pltpu.make_async_copy(e0f.at[pl.ds(0, K)], sidx[slot], semi[slot]).wait()
        pltpu.make_async_copy(e0f.at[pl.ds(0, K)], didx[slot], semi[slot]).wait()

    def g_issue(w, slot):
        def go(ef, h, off):
            pltpu.async_copy(h.at[sidx[slot]], rows[slot], semg[slot])
        rel_branch(w, go)

    def g_wait(slot):
        pltpu.make_async_copy(h0.at[pl.ds(0, K)], rows[slot], semg[slot]).wait()

    def scatter(slot):
        pltpu.sync_copy(rows[slot], acc.at[didx[slot]], add=True)

    zbase = s * RPT
    for w in range(UNROLL):
        idx_issue(w, w)

    _TAIL = N - (NS - 1) * RPT
    use_root = jnp.where(c == 0, s < NS // 2, s >= NS // 2)

    @pl.when(jnp.logical_and(use_root, s < NS - 1))
    def _():
        pltpu.async_copy(xroot.at[pl.ds(zbase, RPT), :],
                         acc.at[pl.ds(zbase, RPT), :], semg0).wait()

    @pl.when(jnp.logical_and(use_root, s == NS - 1))
    def _():
        pltpu.async_copy(xroot.at[pl.ds(zbase, _TAIL), :],
                         acc.at[pl.ds(zbase, _TAIL), :], semg0).wait()

    @pl.when(jnp.logical_not(use_root))
    def _():
        z16 = jnp.zeros((16,), jnp.float32)

        def _zrow(i, carry):
            for j in range(D // 16):
                zbuf[i, pl.ds(j * 16, 16)] = z16
            return carry

        lax.fori_loop(0, ZR, _zrow, 0)
        for i in range(RPT // ZR):
            pltpu.sync_copy(zbuf, acc.at[pl.ds(zbase + i * ZR, ZR), :])

    plsc.subcore_barrier()

    idx_wait(0)
    g_issue(0, 0)
    idx_wait(1)
    g_issue(1, 1)

    def _body(w3, carry):
        w = UNROLL * w3
        for j in range(UNROLL):
            jn = (j + 2) % UNROLL
            g_wait(j)
            scatter(j)

            @pl.when(w + j + UNROLL < NWIN)
            def _():
                idx_issue(w + j + UNROLL, j)

            @pl.when(w + j + 2 < NWIN)
            def _():
                idx_wait(jn)
                g_issue(w + j + 2, jn)

        return carry

    lax.fori_loop(0, NWIN // UNROLL, _body, 0)
    plsc.subcore_barrier()

    obase = s * RPT
    out = (out0, out1)
    for cc in range(NC):
        @pl.when(jnp.logical_and(c == cc, s < NS - 1))
        def _(cc=cc):
            pltpu.sync_copy(acc.at[pl.ds(obase, RPT), :],
                            out[cc].at[pl.ds(obase, RPT), :])

        @pl.when(jnp.logical_and(c == cc, s == NS - 1))
        def _(cc=cc):
            pltpu.sync_copy(acc.at[pl.ds(obase, _TAIL), :],
                            out[cc].at[pl.ds(obase, _TAIL), :])


def _make_sc_call():
    return pl.kernel(
        _sc_body,
        out_type=(jax.ShapeDtypeStruct((NP, D), jnp.float32),) * 2,
        mesh=plsc.VectorSubcoreMesh(core_axis_name="c", subcore_axis_name="s"),
        scratch_types=[
            pltpu.VMEM_SHARED((NP, D), jnp.float32),
        ]
        + [pltpu.VMEM((K,), jnp.int32)] * 6
        + [pltpu.VMEM((K, D), jnp.float32)] * 3
        + [pltpu.VMEM((ZR, D), jnp.float32)]
        + [pltpu.SemaphoreType.DMA] * 6,
    )


def kernel(x, edge_index_0, edge_index_1, edge_index_2, W0, W1, W2, W_root, b_root):
    h0, h1, h2, xroot = _mm_call(x, W0, W1, W2, W_root, b_root.reshape(1, D))
    sc = _make_sc_call()
    p0, p1 = sc(edge_index_0.reshape(2 * E), edge_index_1.reshape(2 * E),
                edge_index_2.reshape(2 * E), h0, h1, h2, xroot)
    return _combine_call(p0, p1)
